# baseline (speedup 1.0000x reference)
import functools

import jax
import jax.numpy as jnp
from jax import lax
from jax.experimental import pallas as pl
from jax.experimental.pallas import tpu as pltpu
from jax.experimental.pallas import tpu_sc as plsc

_D_MODEL = 2112
_MAX_LENGTH = 2048
_EMBED_DIM = _D_MODEL - _MAX_LENGTH


def _make_sc_gather(B, D):
    info = plsc.get_sparse_core_info()
    NC, NS = info.num_cores, info.num_subcores
    NW = NC * NS
    assert B % (8 * NW) == 0
    b_per_w = B // NW
    mesh = plsc.VectorSubcoreMesh(core_axis_name="c", subcore_axis_name="s")

    @functools.partial(
        pl.kernel,
        mesh=mesh,
        out_type=jax.ShapeDtypeStruct((B, D), jnp.float32),
        scratch_types=[
            pltpu.VMEM((b_per_w,), jnp.int32),
            pltpu.VMEM((b_per_w, D), jnp.float32),
            pltpu.SemaphoreType.DMA,
        ],
        compiler_params=pltpu.CompilerParams(use_tc_tiling_on_sc=False),
    )
    def gather_kernel(table_hbm, idx_hbm, out_hbm, idx_v, rows_v, sem):
        wid = lax.axis_index("s") * NC + lax.axis_index("c")
        base = wid * b_per_w
        pltpu.sync_copy(idx_hbm.at[pl.ds(base, b_per_w)], idx_v)
        pltpu.async_copy(table_hbm.at[idx_v], rows_v, sem).wait()
        pltpu.sync_copy(rows_v, out_hbm.at[pl.ds(base, b_per_w)])

    return gather_kernel


def _tc_body(tok_ref, pos_ref, out_ref):
    rows = tok_ref.shape[0]
    col = lax.broadcasted_iota(jnp.int32, (rows, _MAX_LENGTH), 1)
    pe = (col == pos_ref[...]).astype(jnp.float32)
    out_ref[...] = jnp.concatenate([tok_ref[...], pe], axis=1)


def _tc_fuse(tok, pos2d, B, rows_per_block):
    grid = (B // rows_per_block,)
    return pl.pallas_call(
        _tc_body,
        grid=grid,
        in_specs=[
            pl.BlockSpec((rows_per_block, _EMBED_DIM), lambda i: (i, 0)),
            pl.BlockSpec((rows_per_block, 1), lambda i: (i, 0)),
        ],
        out_specs=pl.BlockSpec((rows_per_block, _D_MODEL), lambda i: (i, 0)),
        out_shape=jax.ShapeDtypeStruct((B, _D_MODEL), jnp.float32),
    )(tok, pos2d)


def kernel(x, pos, token_embed_weight):
    batch, seq = x.shape
    B = batch * seq
    x_flat = x.reshape(B).astype(jnp.int32)
    pos2d = pos.reshape(B, 1).astype(jnp.int32)
    tok = _make_sc_gather(B, _EMBED_DIM)(token_embed_weight, x_flat)
    out = _tc_fuse(tok, pos2d, B, rows_per_block=512)
    return out.reshape(batch, seq, _D_MODEL)


# speedup vs baseline: 4.9605x; 4.9605x over previous
import functools

import jax
import jax.numpy as jnp
from jax import lax
from jax.experimental import pallas as pl
from jax.experimental.pallas import tpu as pltpu
from jax.experimental.pallas import tpu_sc as plsc

_D_MODEL = 2112
_MAX_LENGTH = 2048
_EMBED_DIM = _D_MODEL - _MAX_LENGTH
_LANES = 128
_NSLOT = 8


def _make_sc_gather(B, D):
    info = plsc.get_sparse_core_info()
    NC, NS = info.num_cores, info.num_subcores
    NW = NC * NS
    b_per_w = B // NW
    n_blocks = b_per_w // 16
    mesh = plsc.VectorSubcoreMesh(core_axis_name="c", subcore_axis_name="s")

    @functools.partial(
        pl.kernel,
        mesh=mesh,
        out_type=jax.ShapeDtypeStruct((B, D), jnp.float32),
        scratch_types=[
            pltpu.VMEM((b_per_w,), jnp.int32),
            pltpu.VMEM((_NSLOT, D, _LANES), jnp.float32),
            pltpu.VMEM((b_per_w, D), jnp.float32),
            pltpu.SemaphoreType.DMA((_NSLOT,)),
        ],
        compiler_params=pltpu.CompilerParams(needs_layout_passes=False),
    )
    def gather_kernel(wt_hbm, idx_hbm, out_hbm, idx_v, grp_v, crow_v, sem):
        wid = lax.axis_index("s") * NC + lax.axis_index("c")
        base = wid * b_per_w
        pltpu.sync_copy(idx_hbm.at[pl.ds(base, b_per_w)], idx_v)

        def fire(xv, slot):
            col0 = pl.multiple_of((xv >> 7) * _LANES, _LANES)
            pltpu.async_copy(
                wt_hbm.at[:, pl.ds(col0, _LANES)],
                grp_v.at[slot], sem.at[slot])

        def drain(slot):
            pltpu.make_async_copy(
                wt_hbm.at[:, pl.ds(0, _LANES)],
                grp_v.at[slot], sem.at[slot]).wait()

        def extract(xv, slot, row):
            lane = jnp.broadcast_to(xv & (_LANES - 1), (16,))
            for k in range(D // 16):
                rows16 = lax.iota(jnp.int32, 16) + (16 * k)
                v = plsc.load_gather(grp_v.at[slot], [rows16, lane])
                crow_v[row, pl.ds(16 * k, 16)] = v

        def block(c, pvec):
            vec = idx_v[pl.ds(c * 16, 16)]
            for j in range(16):
                slot = j % _NSLOT
                prev = vec[j - 8] if j >= 8 else pvec[j + 8]

                if j >= 8:
                    drain(slot)
                    extract(prev, slot, c * 16 + j - 8)
                else:
                    @pl.when(c > 0)
                    def _():
                        drain(slot)
                        extract(prev, slot, c * 16 + j - 8)
                fire(vec[j], slot)
            return vec

        lvec = pl.loop(0, n_blocks, init_carry=idx_v[pl.ds(0, 16)])(block)
        for j in range(8):
            drain(j)
            extract(lvec[j + 8], j, b_per_w - 8 + j)
        pltpu.sync_copy(crow_v, out_hbm.at[pl.ds(base, b_per_w)])

    return gather_kernel


_CB = 528


def _tc_body(tok_ref, pos_ref, out_ref):
    j = pl.program_id(1)
    chan = lax.broadcasted_iota(jnp.int32, (_CB, _MAX_LENGTH), 0)
    target = chan + (j * _CB - _EMBED_DIM)
    out_ref[0] = (target == pos_ref[0]).astype(jnp.float32)

    @pl.when(j == 0)
    def _():
        row = lax.broadcasted_iota(jnp.int32, (_EMBED_DIM, _EMBED_DIM), 0)
        col = lax.broadcasted_iota(jnp.int32, (_EMBED_DIM, _EMBED_DIM), 1)
        eye = (row == col).astype(jnp.float32)
        tok_t = lax.dot_general(eye, tok_ref[...],
                                (((1,), (1,)), ((), ())),
                                preferred_element_type=jnp.float32)
        out_ref[0, 0:_EMBED_DIM, :] = tok_t


def _tc_fuse(tok, pos3, batch, seq):
    grid = (batch, _D_MODEL // _CB)
    return pl.pallas_call(
        _tc_body,
        grid=grid,
        in_specs=[
            pl.BlockSpec((seq, _EMBED_DIM), lambda b, j: (b, 0)),
            pl.BlockSpec((1, 1, seq), lambda b, j: (b, 0, 0)),
        ],
        out_specs=pl.BlockSpec((1, _CB, seq), lambda b, j: (b, j, 0)),
        out_shape=jax.ShapeDtypeStruct((batch, _D_MODEL, seq), jnp.float32),
    )(tok, pos3)


def kernel(x, pos, token_embed_weight):
    batch, seq = x.shape
    B = batch * seq
    x_flat = x.reshape(B).astype(jnp.int32)
    pos3 = pos.reshape(batch, 1, seq).astype(jnp.int32)
    w_t = token_embed_weight.T
    tok = _make_sc_gather(B, _EMBED_DIM)(w_t, x_flat)
    out_t = _tc_fuse(tok, pos3, batch, seq)
    return jnp.swapaxes(out_t, 1, 2)


# speedup vs baseline: 5.0373x; 1.0155x over previous
import functools

import jax
import jax.numpy as jnp
from jax import lax
from jax.experimental import pallas as pl
from jax.experimental.pallas import tpu as pltpu
from jax.experimental.pallas import tpu_sc as plsc

_D_MODEL = 2112
_MAX_LENGTH = 2048
_EMBED_DIM = _D_MODEL - _MAX_LENGTH
_LANES = 128
_NSLOT = 8


def _make_sc_gather(B, D):
    info = plsc.get_sparse_core_info()
    NC, NS = info.num_cores, info.num_subcores
    NW = NC * NS
    b_per_w = B // NW
    n_blocks = b_per_w // 16
    mesh = plsc.VectorSubcoreMesh(core_axis_name="c", subcore_axis_name="s")

    @functools.partial(
        pl.kernel,
        mesh=mesh,
        out_type=jax.ShapeDtypeStruct((B, D), jnp.float32),
        scratch_types=[
            pltpu.VMEM((b_per_w,), jnp.int32),
            pltpu.VMEM((_NSLOT, D, _LANES), jnp.float32),
            pltpu.VMEM((b_per_w, D), jnp.float32),
            pltpu.SemaphoreType.DMA((_NSLOT,)),
        ],
        compiler_params=pltpu.CompilerParams(needs_layout_passes=False),
    )
    def gather_kernel(wt_hbm, idx_hbm, out_hbm, idx_v, grp_v, crow_v, sem):
        wid = lax.axis_index("s") * NC + lax.axis_index("c")
        base = wid * b_per_w
        pltpu.sync_copy(idx_hbm.at[pl.ds(base, b_per_w)], idx_v)

        def fire(xv, slot):
            col0 = pl.multiple_of((xv >> 7) * _LANES, _LANES)
            pltpu.async_copy(
                wt_hbm.at[:, pl.ds(col0, _LANES)],
                grp_v.at[slot], sem.at[slot])

        def drain(slot):
            pltpu.make_async_copy(
                wt_hbm.at[:, pl.ds(0, _LANES)],
                grp_v.at[slot], sem.at[slot]).wait()

        def extract(xv, slot, row):
            lane = jnp.broadcast_to(xv & (_LANES - 1), (16,))
            for k in range(D // 16):
                rows16 = lax.iota(jnp.int32, 16) + (16 * k)
                v = plsc.load_gather(grp_v.at[slot], [rows16, lane])
                crow_v[row, pl.ds(16 * k, 16)] = v

        def block(c, pvec):
            vec = idx_v[pl.ds(c * 16, 16)]
            for j in range(16):
                slot = j % _NSLOT
                prev = vec[j - 8] if j >= 8 else pvec[j + 8]

                if j >= 8:
                    drain(slot)
                    extract(prev, slot, c * 16 + j - 8)
                else:
                    @pl.when(c > 0)
                    def _():
                        drain(slot)
                        extract(prev, slot, c * 16 + j - 8)
                fire(vec[j], slot)
            return vec

        lvec = pl.loop(0, n_blocks, init_carry=idx_v[pl.ds(0, 16)])(block)
        for j in range(8):
            drain(j)
            extract(lvec[j + 8], j, b_per_w - 8 + j)
        pltpu.sync_copy(crow_v, out_hbm.at[pl.ds(base, b_per_w)])

    return gather_kernel


_CB = 528


def _pe_body(pos_ref, out_ref):
    j = pl.program_id(1)
    chan = lax.broadcasted_iota(jnp.int32, (_CB, _MAX_LENGTH), 0)
    target = chan + (j * _CB - _EMBED_DIM)
    out_ref[0] = (target == pos_ref[0]).astype(jnp.float32)


def _pe_write(pos3, batch, seq):
    grid = (batch, _D_MODEL // _CB)
    return pl.pallas_call(
        _pe_body,
        grid=grid,
        in_specs=[pl.BlockSpec((1, 1, seq), lambda b, j: (b, 0, 0))],
        out_specs=pl.BlockSpec((1, _CB, seq), lambda b, j: (b, j, 0)),
        out_shape=jax.ShapeDtypeStruct((batch, _D_MODEL, seq), jnp.float32),
    )(pos3)


def _tok_body(pe_ref, tok_ref, out_ref):
    del pe_ref
    row = lax.broadcasted_iota(jnp.int32, (_EMBED_DIM, _EMBED_DIM), 0)
    col = lax.broadcasted_iota(jnp.int32, (_EMBED_DIM, _EMBED_DIM), 1)
    eye = (row == col).astype(jnp.float32)
    out_ref[0] = lax.dot_general(eye, tok_ref[...],
                                 (((1,), (1,)), ((), ())),
                                 preferred_element_type=jnp.float32)


def _tok_write(pe, tok, batch, seq):
    return pl.pallas_call(
        _tok_body,
        grid=(batch,),
        in_specs=[
            pl.BlockSpec(memory_space=pl.ANY),
            pl.BlockSpec((seq, _EMBED_DIM), lambda b: (b, 0)),
        ],
        out_specs=pl.BlockSpec((1, _EMBED_DIM, seq), lambda b: (b, 0, 0)),
        out_shape=jax.ShapeDtypeStruct((batch, _D_MODEL, seq), jnp.float32),
        input_output_aliases={0: 0},
    )(pe, tok)


def kernel(x, pos, token_embed_weight):
    batch, seq = x.shape
    B = batch * seq
    x_flat = x.reshape(B).astype(jnp.int32)
    pos3 = pos.reshape(batch, 1, seq).astype(jnp.int32)
    w_t = token_embed_weight.T
    tok = _make_sc_gather(B, _EMBED_DIM)(w_t, x_flat)
    pe = _pe_write(pos3, batch, seq)
    out_t = _tok_write(pe, tok, batch, seq)
    return jnp.swapaxes(out_t, 1, 2)
